# Initial kernel scaffold; baseline (speedup 1.0000x reference)
import jax
import jax.numpy as jnp
from jax.experimental import pallas as pl


def kernel(x, q_values, quantiles):
    raise NotImplementedError("write your pallas kernel here")


# speedup vs baseline: 67.2688x; 67.2688x over previous
import jax
import jax.numpy as jnp
from jax import lax
from jax.experimental import pallas as pl
from jax.experimental.pallas import tpu as pltpu
from jax.experimental.pallas import tpu_sc as plsc

_NC = 2
_NS = 16
_L = 16
_NW = _NC * _NS


def _qnorm_body(x_hbm, tab_hbm, col_hbm, q_hbm, out_hbm, xv, tv, cv, qv, ov):
    wid = lax.axis_index("s") * _NC + lax.axis_index("c")
    npt = xv.shape[0]
    nq = q_hbm.shape[0]
    base = wid * npt
    pltpu.sync_copy(x_hbm.at[pl.ds(base, npt)], xv)
    pltpu.sync_copy(tab_hbm, tv)
    pltpu.sync_copy(col_hbm, cv)
    pltpu.sync_copy(q_hbm, qv)

    def body(i, carry):
        off = i * _L
        x16 = xv[pl.ds(off, _L)]
        tix = cv[pl.ds(off, _L)]
        pos = jnp.zeros((_L,), jnp.int32)
        k = nq // 2
        while k >= 1:
            v = plsc.load_gather(tv, [tix + (pos + (k - 1))])
            pos = jnp.where(v <= x16, pos + k, pos)
            k //= 2
        r = jnp.minimum(jnp.maximum(pos - 1, 0), nq - 2)
        low = plsc.load_gather(tv, [tix + r])
        high = plsc.load_gather(tv, [tix + r + 1])
        ql = plsc.load_gather(qv, [r])
        qh = plsc.load_gather(qv, [r + 1])
        t = (x16 - low) / (high - low + 1e-9)
        ov[pl.ds(off, _L)] = ql + t * (qh - ql)
        return carry

    lax.fori_loop(0, npt // _L, body, 0)
    pltpu.sync_copy(ov, out_hbm.at[pl.ds(base, npt)])


def kernel(x, q_values, quantiles):
    b, f = x.shape
    nq = q_values.shape[0]
    n = b * f
    npt = n // _NW
    xf = x.reshape(-1)
    tab = q_values.T.reshape(-1)
    col = (jnp.arange(npt, dtype=jnp.int32) % f) * nq
    mesh = plsc.VectorSubcoreMesh(core_axis_name="c", subcore_axis_name="s")
    out = pl.kernel(
        _qnorm_body,
        out_type=jax.ShapeDtypeStruct((n,), jnp.float32),
        mesh=mesh,
        compiler_params=pltpu.CompilerParams(needs_layout_passes=False),
        scratch_types=[
            pltpu.VMEM((npt,), jnp.float32),
            pltpu.VMEM((f * nq,), jnp.float32),
            pltpu.VMEM((npt,), jnp.int32),
            pltpu.VMEM((nq,), jnp.float32),
            pltpu.VMEM((npt,), jnp.float32),
        ],
    )(xf, tab, col, quantiles)
    return out.reshape(b, f)


# speedup vs baseline: 93.5504x; 1.3907x over previous
import jax
import jax.numpy as jnp
from jax import lax
from jax.experimental import pallas as pl
from jax.experimental.pallas import tpu as pltpu
from jax.experimental.pallas import tpu_sc as plsc

_NC = 2
_NS = 16
_L = 16
_NW = _NC * _NS


def _qnorm_body(x_hbm, tab_hbm, col_hbm, q_hbm, out_hbm, xv, tv, cv, qv, ov):
    wid = lax.axis_index("s") * _NC + lax.axis_index("c")
    npt = xv.shape[0]
    nq = q_hbm.shape[0]
    base = wid * npt
    pltpu.sync_copy(x_hbm.at[pl.ds(base, npt)], xv)
    pltpu.sync_copy(tab_hbm, tv)
    pltpu.sync_copy(col_hbm, cv)
    pltpu.sync_copy(q_hbm, qv)

    @plsc.parallel_loop(0, npt // _L, 1, unroll=4)
    def body(i):
        off = i * _L
        x16 = xv[pl.ds(off, _L)]
        tix = cv[pl.ds(off, _L)]
        pos = jnp.zeros((_L,), jnp.int32)
        k = nq // 2
        while k >= 1:
            v = plsc.load_gather(tv, [tix + (pos + (k - 1))])
            pos = jnp.where(v <= x16, pos + k, pos)
            k //= 2
        r = jnp.minimum(jnp.maximum(pos - 1, 0), nq - 2)
        low = plsc.load_gather(tv, [tix + r])
        high = plsc.load_gather(tv, [tix + r + 1])
        ql = plsc.load_gather(qv, [r])
        qh = plsc.load_gather(qv, [r + 1])
        t = (x16 - low) / (high - low + 1e-9)
        ov[pl.ds(off, _L)] = ql + t * (qh - ql)

    pltpu.sync_copy(ov, out_hbm.at[pl.ds(base, npt)])


def kernel(x, q_values, quantiles):
    b, f = x.shape
    nq = q_values.shape[0]
    n = b * f
    npt = n // _NW
    xf = x.reshape(-1)
    tab = q_values.T.reshape(-1)
    col = (jnp.arange(npt, dtype=jnp.int32) % f) * nq
    mesh = plsc.VectorSubcoreMesh(core_axis_name="c", subcore_axis_name="s")
    out = pl.kernel(
        _qnorm_body,
        out_type=jax.ShapeDtypeStruct((n,), jnp.float32),
        mesh=mesh,
        compiler_params=pltpu.CompilerParams(needs_layout_passes=False),
        scratch_types=[
            pltpu.VMEM((npt,), jnp.float32),
            pltpu.VMEM((f * nq,), jnp.float32),
            pltpu.VMEM((npt,), jnp.int32),
            pltpu.VMEM((nq,), jnp.float32),
            pltpu.VMEM((npt,), jnp.float32),
        ],
    )(xf, tab, col, quantiles)
    return out.reshape(b, f)
